# baseline (speedup 1.0000x reference)
import functools

import jax
import jax.numpy as jnp
from jax import lax
from jax.experimental import pallas as pl
from jax.experimental.pallas import tpu as pltpu
from jax.experimental.pallas import tpu_sc as plsc

N = 10000
NP = 10240
E = 320000
D = 128
NEG_SLOPE = 0.2
BN = 1024
GRID = NP // BN

CHUNK = 2000
NCH = E // CHUNK
GRP = CHUNK // 16


def _ln(r, g, b):
    m = jnp.mean(r, axis=0, keepdims=True)
    v = jnp.mean((r - m) * (r - m), axis=0, keepdims=True)
    return (r - m) / jnp.sqrt(v + 1e-5) * g + b


def _prelu(x, a):
    return jnp.where(x >= 0, x, a * x)


def _dot(a, b, dims):
    return lax.dot_general(a, b, (dims, ((), ())),
                           preferred_element_type=jnp.float32)


def _mlp_core_t(x, wi, bi, wr1, br1, g, bl, a1, wr2, br2):
    t0 = _dot(wi, x, ((1,), (0,))) + bi
    r = _dot(wr1, t0, ((1,), (0,))) + br1
    r = _ln(r, g, bl)
    r = _prelu(r, a1)
    r = _dot(wr2, r, ((1,), (0,))) + br2
    return t0 + r


def _full(shape):
    return pl.BlockSpec(shape, lambda i: (0,) * len(shape))


def _mlp_in_body(x_ref, wi, bi, wr1, br1, g, bl, a1, wr2, br2, wo, bo, a2,
                 out_ref):
    x = x_ref[...]
    t0 = _dot(wi[...], x, ((1,), (1,))) + bi[...]
    r = _dot(wr1[...], t0, ((1,), (0,))) + br1[...]
    r = _ln(r, g[...], bl[...])
    r = _prelu(r, a1[...])
    r = _dot(wr2[...], r, ((1,), (0,))) + br2[...]
    t0 = t0 + r
    o = _dot(wo[...], t0, ((1,), (0,))) + bo[...]
    out_ref[...] = _prelu(o, a2[...])


def _mlp_in(x, p):
    dh, din, dout = 256, 128, 128
    specs = [
        pl.BlockSpec((BN, din), lambda i: (i, 0)),
        _full((dh, din)), _full((dh, 1)),
        _full((dh, dh)), _full((dh, 1)),
        _full((dh, 1)), _full((dh, 1)), _full((1, 1)),
        _full((dh, dh)), _full((dh, 1)),
        _full((dout, dh)), _full((dout, 1)), _full((1, 1)),
    ]
    fn = pl.pallas_call(
        _mlp_in_body,
        grid=(GRID,),
        in_specs=specs,
        out_specs=pl.BlockSpec((dout, BN), lambda i: (0, i)),
        out_shape=jax.ShapeDtypeStruct((dout, NP), jnp.float32),
    )
    return fn(x, p['in_w'], p['in_b'].reshape(dh, 1),
              p['r1_w'], p['r1_b'].reshape(dh, 1),
              p['ln_g'].reshape(dh, 1), p['ln_b'].reshape(dh, 1),
              p['a1'].reshape(1, 1),
              p['r2_w'], p['r2_b'].reshape(dh, 1),
              p['out_w'], p['out_b'].reshape(dout, 1),
              p['a2'].reshape(1, 1))


def _mlp_mid_body(a_ref, bc_ref, wi, bi, wr1, br1, g, bl, a1, wr2, br2,
                  wo, bo, a2, out_ref):
    x = a_ref[...] + bc_ref[...]
    t0 = _mlp_core_t(x, wi[...], bi[...], wr1[...], br1[...], g[...],
                     bl[...], a1[...], wr2[...], br2[...])
    o = _dot(wo[...], t0, ((1,), (0,))) + bo[...]
    out_ref[...] = _prelu(o, a2[...])


def _mlp_mid(aT, bcolv, p):
    dh = 128
    specs = [
        pl.BlockSpec((D, BN), lambda i: (0, i)),
        _full((D, 1)),
        _full((dh, D)), _full((dh, 1)),
        _full((dh, dh)), _full((dh, 1)),
        _full((dh, 1)), _full((dh, 1)), _full((1, 1)),
        _full((dh, dh)), _full((dh, 1)),
        _full((dh, dh)), _full((dh, 1)), _full((1, 1)),
    ]
    fn = pl.pallas_call(
        _mlp_mid_body,
        grid=(GRID,),
        in_specs=specs,
        out_specs=pl.BlockSpec((dh, BN), lambda i: (0, i)),
        out_shape=jax.ShapeDtypeStruct((dh, NP), jnp.float32),
    )
    return fn(aT, bcolv,
              p['in_w'], p['in_b'].reshape(dh, 1),
              p['r1_w'], p['r1_b'].reshape(dh, 1),
              p['ln_g'].reshape(dh, 1), p['ln_b'].reshape(dh, 1),
              p['a1'].reshape(1, 1),
              p['r2_w'], p['r2_b'].reshape(dh, 1),
              p['out_w'], p['out_b'].reshape(dh, 1),
              p['a2'].reshape(1, 1))


def _make_prelude(n_in, H):
    def body(*refs):
        if n_in == 2:
            a0, a1v, bc, fw, al, ar, f_ref, el_ref, er_ref = refs
            h = a0[...] + a1v[...] + bc[...]
        else:
            a0, bc, fw, al, ar, f_ref, el_ref, er_ref = refs
            h = a0[...] + bc[...]
        feat = _dot(fw[...], h, ((1,), (0,)))
        f_ref[...] = feat
        el_ref[...] = _dot(al[...], feat, ((1,), (0,)))
        er_ref[...] = _dot(ar[...], feat, ((1,), (0,)))

    specs = [pl.BlockSpec((D, BN), lambda i: (0, i))] * n_in + [
        _full((D, 1)), _full((D, D)), _full((H, D)), _full((H, D))]
    return pl.pallas_call(
        body,
        grid=(GRID,),
        in_specs=specs,
        out_specs=[pl.BlockSpec((D, BN), lambda i: (0, i)),
                   pl.BlockSpec((H, BN), lambda i: (0, i)),
                   pl.BlockSpec((H, BN), lambda i: (0, i))],
        out_shape=[jax.ShapeDtypeStruct((D, NP), jnp.float32),
                   jax.ShapeDtypeStruct((H, NP), jnp.float32),
                   jax.ShapeDtypeStruct((H, NP), jnp.float32)],
    )


def _block_diag_attn(a):
    Hh, dh = a.shape
    rows = jnp.arange(Hh)[:, None] * jnp.ones((1, dh), jnp.int32)
    cols = jnp.arange(Hh)[:, None] * dh + jnp.arange(dh)[None, :]
    return jnp.zeros((Hh, D), jnp.float32).at[rows, cols].set(a)


def _make_gat_sc(H):
    mesh = plsc.VectorSubcoreMesh(core_axis_name="c", subcore_axis_name="s",
                                  num_cores=2, num_subcores=16)

    @functools.partial(
        pl.kernel,
        out_type=jax.ShapeDtypeStruct((D, NP), jnp.float32),
        mesh=mesh,
        compiler_params=pltpu.CompilerParams(needs_layout_passes=False),
        scratch_types=[
            pltpu.VMEM((4, NP), jnp.float32),
            pltpu.VMEM((NP,), jnp.float32),
            pltpu.VMEM((NP,), jnp.float32),
            pltpu.VMEM((NP,), jnp.float32),
            pltpu.VMEM((4, NP), jnp.float32),
            pltpu.VMEM((CHUNK,), jnp.int32),
            pltpu.VMEM((CHUNK,), jnp.int32),
            pltpu.VMEM((CHUNK,), jnp.int32),
            pltpu.VMEM((CHUNK,), jnp.int32),
            pltpu.SemaphoreType.DMA,
            pltpu.SemaphoreType.DMA,
            pltpu.SemaphoreType.DMA,
            pltpu.SemaphoreType.DMA,
        ],
    )
    def k(featT, el, er, src, dst, out, ft, elv, erv, denom, acc,
          sv0, dv0, sv1, dv1, sm0, sm1, sm2, sm3):
        t = lax.axis_index("s") * 2 + lax.axis_index("c")
        h = t // 8 if H == 4 else 0
        pltpu.sync_copy(featT.at[pl.ds(4 * t, 4), :], ft)
        pltpu.sync_copy(el.at[h], elv)
        pltpu.sync_copy(er.at[h], erv)

        zero = jnp.zeros((16,), jnp.float32)

        @plsc.parallel_loop(0, NP // 16, unroll=8)
        def _(i):
            denom[pl.ds(i * 16, 16)] = zero
            for c in range(4):
                acc[c, pl.ds(i * 16, 16)] = zero

        ccs = [jnp.full((16,), c, jnp.int32) for c in range(4)]

        def mk_fetch(ci, svb, dvb, ss, sd):
            c = jnp.minimum(ci, NCH - 1)
            return (pltpu.make_async_copy(src.at[pl.ds(c * CHUNK, CHUNK)],
                                          svb, ss),
                    pltpu.make_async_copy(dst.at[pl.ds(c * CHUNK, CHUNK)],
                                          dvb, sd))

        def start(ci, svb, dvb, ss, sd):
            a, b = mk_fetch(ci, svb, dvb, ss, sd)
            a.start()
            b.start()

        def wait(ci, svb, dvb, ss, sd):
            a, b = mk_fetch(ci, svb, dvb, ss, sd)
            a.wait()
            b.wait()

        def process(svb, dvb):
            @plsc.parallel_loop(0, GRP, unroll=25)
            def _(gi):
                s16 = svb[pl.ds(gi * 16, 16)]
                d16 = dvb[pl.ds(gi * 16, 16)]
                els = plsc.load_gather(elv, [s16])
                erd = plsc.load_gather(erv, [d16])
                e = els + erd
                e = jnp.where(e >= 0, e, NEG_SLOPE * e)
                ex = jnp.exp(e)
                plsc.addupdate_scatter(denom, [d16], ex)
                for c in range(4):
                    fv = plsc.load_gather(ft, [ccs[c], s16])
                    plsc.addupdate_scatter(acc, [ccs[c], d16], fv * ex)

        start(0, sv0, dv0, sm0, sm1)
        start(1, sv1, dv1, sm2, sm3)

        def pair_body(kk, _):
            ci = kk * 2
            wait(ci, sv0, dv0, sm0, sm1)
            process(sv0, dv0)
            start(ci + 2, sv0, dv0, sm0, sm1)
            wait(ci + 1, sv1, dv1, sm2, sm3)
            process(sv1, dv1)
            start(ci + 3, sv1, dv1, sm2, sm3)
            return 0
        lax.fori_loop(0, NCH // 2, pair_body, 0)
        wait(NCH - 1, sv0, dv0, sm0, sm1)
        wait(NCH - 1, sv1, dv1, sm2, sm3)

        one = jnp.ones((16,), jnp.float32)

        @plsc.parallel_loop(0, NP // 16, unroll=8)
        def _(i):
            dn = denom[pl.ds(i * 16, 16)]
            rd = jnp.where(dn > 0, one / dn, zero)
            for c in range(4):
                acc[c, pl.ds(i * 16, 16)] = acc[c, pl.ds(i * 16, 16)] * rd
        pltpu.sync_copy(acc, out.at[pl.ds(4 * t, 4), :])

    return k


def _final_body(u0_ref, u1_ref, b0, b1, eye, wi, bi, wr1, br1, g, bl, a1,
                wr2, br2, wo, bo_r, a2, out_ref, cat_ref):
    u0 = u0_ref[...] + b0[...]
    u1 = u1_ref[...] + b1[...]
    t0 = _mlp_core_t(u1, wi[...], bi[...], wr1[...], br1[...], g[...],
                     bl[...], a1[...], wr2[...], br2[...])
    o = _dot(t0, wo[...], ((0,), (1,))) + bo_r[...]
    out_ref[...] = _prelu(o, a2[...])
    ident = eye[...]
    u0_rm = _dot(u0, ident, ((0,), (0,)))
    u1_rm = _dot(u1, ident, ((0,), (0,)))
    cat_ref[...] = jnp.concatenate([u0_rm, u1_rm], axis=1)


def _final(a_u0, a_u1, b0col, b1col, p):
    dh = 128
    specs = [
        pl.BlockSpec((D, BN), lambda i: (0, i)),
        pl.BlockSpec((D, BN), lambda i: (0, i)),
        _full((D, 1)), _full((D, 1)), _full((D, D)),
        _full((dh, D)), _full((dh, 1)),
        _full((dh, dh)), _full((dh, 1)),
        _full((dh, 1)), _full((dh, 1)), _full((1, 1)),
        _full((dh, dh)), _full((dh, 1)),
        _full((dh, dh)), _full((1, dh)), _full((1, 1)),
    ]
    fn = pl.pallas_call(
        _final_body,
        grid=(GRID,),
        in_specs=specs,
        out_specs=[pl.BlockSpec((BN, dh), lambda i: (i, 0)),
                   pl.BlockSpec((BN, 2 * D), lambda i: (i, 0))],
        out_shape=[jax.ShapeDtypeStruct((NP, dh), jnp.float32),
                   jax.ShapeDtypeStruct((NP, 2 * D), jnp.float32)],
    )
    return fn(a_u0, a_u1, b0col, b1col, jnp.eye(D, dtype=jnp.float32),
              p['in_w'], p['in_b'].reshape(dh, 1),
              p['r1_w'], p['r1_b'].reshape(dh, 1),
              p['ln_g'].reshape(dh, 1), p['ln_b'].reshape(dh, 1),
              p['a1'].reshape(1, 1),
              p['r2_w'], p['r2_b'].reshape(dh, 1),
              p['out_w'], p['out_b'].reshape(1, dh),
              p['a2'].reshape(1, 1))


_prelude_1_4 = _make_prelude(1, 4)
_prelude_2_4 = _make_prelude(2, 4)
_prelude_2_1 = _make_prelude(2, 1)
_gat_sc_4 = _make_gat_sc(4)
_gat_sc_1 = _make_gat_sc(1)


def kernel(x_t, edge_index, time_embed, params):
    del time_embed
    p = params
    x = jnp.pad(x_t, ((0, NP - N), (0, 0)))
    src = edge_index[0]
    dst = edge_index[1]

    def bcol(q):
        return q['bias'].reshape(D, 1)

    def attn(q):
        return (q['fc_w'], _block_diag_attn(q['attn_l']),
                _block_diag_attn(q['attn_r']))

    zcol = jnp.zeros((D, 1), jnp.float32)

    hT = _mlp_in(x, p['mlp_in'])
    f, el, er = _prelude_1_4(hT, zcol, *attn(p['down0']))
    a0 = _gat_sc_4(f, el, er, src, dst)
    f, el, er = _prelude_1_4(a0, bcol(p['down0']), *attn(p['down1']))
    a1 = _gat_sc_4(f, el, er, src, dst)
    mT = _mlp_mid(a1, bcol(p['down1']), p['mlp_mid'])
    f, el, er = _prelude_2_4(mT, a1, bcol(p['down1']), *attn(p['up0']))
    au0 = _gat_sc_4(f, el, er, src, dst)
    bsum = (p['up0']['bias'] + p['down0']['bias']).reshape(D, 1)
    f, el, er = _prelude_2_1(au0, a0, bsum, *attn(p['up1']))
    au1 = _gat_sc_1(f, el, er, src, dst)
    out_full, cat_full = _final(au0, au1, bcol(p['up0']), bcol(p['up1']),
                                p['mlp_out'])
    return out_full[:N], cat_full[:N]


# speedup vs baseline: 1.6596x; 1.6596x over previous
import functools

import jax
import jax.numpy as jnp
from jax import lax
from jax.experimental import pallas as pl
from jax.experimental.pallas import tpu as pltpu
from jax.experimental.pallas import tpu_sc as plsc

N = 10000
NP = 10240
E = 320000
D = 128
NEG_SLOPE = 0.2
BN = 1024
GRID = NP // BN

CHUNK = 2000
NCH = E // CHUNK
GRP = CHUNK // 16


def _ln(r, g, b):
    m = jnp.mean(r, axis=0, keepdims=True)
    v = jnp.mean((r - m) * (r - m), axis=0, keepdims=True)
    return (r - m) / jnp.sqrt(v + 1e-5) * g + b


def _prelu(x, a):
    return jnp.where(x >= 0, x, a * x)


def _dot(a, b, dims):
    return lax.dot_general(a, b, (dims, ((), ())),
                           preferred_element_type=jnp.float32)


def _mlp_core_t(x, wi, bi, wr1, br1, g, bl, a1, wr2, br2):
    t0 = _dot(wi, x, ((1,), (0,))) + bi
    r = _dot(wr1, t0, ((1,), (0,))) + br1
    r = _ln(r, g, bl)
    r = _prelu(r, a1)
    r = _dot(wr2, r, ((1,), (0,))) + br2
    return t0 + r


def _full(shape):
    return pl.BlockSpec(shape, lambda i: (0,) * len(shape))


def _mlp_in_body(x_ref, wi, bi, wr1, br1, g, bl, a1, wr2, br2, wo, bo, a2,
                 out_ref):
    x = x_ref[...]
    t0 = _dot(wi[...], x, ((1,), (1,))) + bi[...]
    r = _dot(wr1[...], t0, ((1,), (0,))) + br1[...]
    r = _ln(r, g[...], bl[...])
    r = _prelu(r, a1[...])
    r = _dot(wr2[...], r, ((1,), (0,))) + br2[...]
    t0 = t0 + r
    o = _dot(wo[...], t0, ((1,), (0,))) + bo[...]
    out_ref[...] = _prelu(o, a2[...])


def _mlp_in(x, p):
    dh, din, dout = 256, 128, 128
    specs = [
        pl.BlockSpec((BN, din), lambda i: (i, 0)),
        _full((dh, din)), _full((dh, 1)),
        _full((dh, dh)), _full((dh, 1)),
        _full((dh, 1)), _full((dh, 1)), _full((1, 1)),
        _full((dh, dh)), _full((dh, 1)),
        _full((dout, dh)), _full((dout, 1)), _full((1, 1)),
    ]
    fn = pl.pallas_call(
        _mlp_in_body,
        grid=(GRID,),
        in_specs=specs,
        out_specs=pl.BlockSpec((dout, BN), lambda i: (0, i)),
        out_shape=jax.ShapeDtypeStruct((dout, NP), jnp.float32),
    )
    return fn(x, p['in_w'], p['in_b'].reshape(dh, 1),
              p['r1_w'], p['r1_b'].reshape(dh, 1),
              p['ln_g'].reshape(dh, 1), p['ln_b'].reshape(dh, 1),
              p['a1'].reshape(1, 1),
              p['r2_w'], p['r2_b'].reshape(dh, 1),
              p['out_w'], p['out_b'].reshape(dout, 1),
              p['a2'].reshape(1, 1))


def _mlp_mid_body(a_ref, bc_ref, wi, bi, wr1, br1, g, bl, a1, wr2, br2,
                  wo, bo, a2, out_ref):
    x = a_ref[...] + bc_ref[...]
    t0 = _mlp_core_t(x, wi[...], bi[...], wr1[...], br1[...], g[...],
                     bl[...], a1[...], wr2[...], br2[...])
    o = _dot(wo[...], t0, ((1,), (0,))) + bo[...]
    out_ref[...] = _prelu(o, a2[...])


def _mlp_mid(aT, bcolv, p):
    dh = 128
    specs = [
        pl.BlockSpec((D, BN), lambda i: (0, i)),
        _full((D, 1)),
        _full((dh, D)), _full((dh, 1)),
        _full((dh, dh)), _full((dh, 1)),
        _full((dh, 1)), _full((dh, 1)), _full((1, 1)),
        _full((dh, dh)), _full((dh, 1)),
        _full((dh, dh)), _full((dh, 1)), _full((1, 1)),
    ]
    fn = pl.pallas_call(
        _mlp_mid_body,
        grid=(GRID,),
        in_specs=specs,
        out_specs=pl.BlockSpec((dh, BN), lambda i: (0, i)),
        out_shape=jax.ShapeDtypeStruct((dh, NP), jnp.float32),
    )
    return fn(aT, bcolv,
              p['in_w'], p['in_b'].reshape(dh, 1),
              p['r1_w'], p['r1_b'].reshape(dh, 1),
              p['ln_g'].reshape(dh, 1), p['ln_b'].reshape(dh, 1),
              p['a1'].reshape(1, 1),
              p['r2_w'], p['r2_b'].reshape(dh, 1),
              p['out_w'], p['out_b'].reshape(dh, 1),
              p['a2'].reshape(1, 1))


def _make_prelude(n_in, H):
    def body(*refs):
        if n_in == 2:
            a0, a1v, bc, fw, al, ar, f_ref, el_ref, er_ref = refs
            h = a0[...] + a1v[...] + bc[...]
        else:
            a0, bc, fw, al, ar, f_ref, el_ref, er_ref = refs
            h = a0[...] + bc[...]
        feat = _dot(fw[...], h, ((1,), (0,)))
        f_ref[...] = feat
        el_ref[...] = _dot(al[...], feat, ((1,), (0,)))
        er_ref[...] = _dot(ar[...], feat, ((1,), (0,)))

    specs = [pl.BlockSpec((D, BN), lambda i: (0, i))] * n_in + [
        _full((D, 1)), _full((D, D)), _full((H, D)), _full((H, D))]
    return pl.pallas_call(
        body,
        grid=(GRID,),
        in_specs=specs,
        out_specs=[pl.BlockSpec((D, BN), lambda i: (0, i)),
                   pl.BlockSpec((H, BN), lambda i: (0, i)),
                   pl.BlockSpec((H, BN), lambda i: (0, i))],
        out_shape=[jax.ShapeDtypeStruct((D, NP), jnp.float32),
                   jax.ShapeDtypeStruct((H, NP), jnp.float32),
                   jax.ShapeDtypeStruct((H, NP), jnp.float32)],
    )


def _block_diag_attn(a):
    Hh, dh = a.shape
    rows = jnp.arange(Hh)[:, None] * jnp.ones((1, dh), jnp.int32)
    cols = jnp.arange(Hh)[:, None] * dh + jnp.arange(dh)[None, :]
    return jnp.zeros((Hh, D), jnp.float32).at[rows, cols].set(a)


def _make_gat_sc(H):
    mesh = plsc.VectorSubcoreMesh(core_axis_name="c", subcore_axis_name="s",
                                  num_cores=2, num_subcores=16)

    @functools.partial(
        pl.kernel,
        out_type=jax.ShapeDtypeStruct((D, NP), jnp.float32),
        mesh=mesh,
        compiler_params=pltpu.CompilerParams(needs_layout_passes=False),
        scratch_types=[
            pltpu.VMEM((4, NP), jnp.float32),
            pltpu.VMEM((NP,), jnp.float32),
            pltpu.VMEM((NP,), jnp.float32),
            pltpu.VMEM((NP,), jnp.float32),
            pltpu.VMEM((4, NP), jnp.float32),
            pltpu.VMEM((CHUNK,), jnp.int32),
            pltpu.VMEM((CHUNK,), jnp.int32),
            pltpu.VMEM((CHUNK,), jnp.int32),
            pltpu.VMEM((CHUNK,), jnp.int32),
            pltpu.SemaphoreType.DMA,
            pltpu.SemaphoreType.DMA,
            pltpu.SemaphoreType.DMA,
            pltpu.SemaphoreType.DMA,
        ],
    )
    def k(featT, el, er, src, dst, out, ft, elv, erv, denom, acc,
          sv0, dv0, sv1, dv1, sm0, sm1, sm2, sm3):
        t = lax.axis_index("s") * 2 + lax.axis_index("c")
        h = t // 8 if H == 4 else 0
        pltpu.sync_copy(featT.at[pl.ds(4 * t, 4), :], ft)
        pltpu.sync_copy(el.at[h], elv)
        pltpu.sync_copy(er.at[h], erv)

        zero = jnp.zeros((16,), jnp.float32)

        @plsc.parallel_loop(0, NP // 16, unroll=8)
        def _(i):
            denom[pl.ds(i * 16, 16)] = zero
            for c in range(4):
                acc[c, pl.ds(i * 16, 16)] = zero

        ccs = [jnp.full((16,), c, jnp.int32) for c in range(4)]

        def mk_fetch(ci, svb, dvb, ss, sd):
            c = jnp.minimum(ci, NCH - 1)
            return (pltpu.make_async_copy(src.at[pl.ds(c * CHUNK, CHUNK)],
                                          svb, ss),
                    pltpu.make_async_copy(dst.at[pl.ds(c * CHUNK, CHUNK)],
                                          dvb, sd))

        def start(ci, svb, dvb, ss, sd):
            a, b = mk_fetch(ci, svb, dvb, ss, sd)
            a.start()
            b.start()

        def wait(ci, svb, dvb, ss, sd):
            a, b = mk_fetch(ci, svb, dvb, ss, sd)
            a.wait()
            b.wait()

        def process(svb, dvb):
            @plsc.parallel_loop(0, GRP, unroll=5)
            def _(gi):
                s16 = svb[pl.ds(gi * 16, 16)]
                d16 = dvb[pl.ds(gi * 16, 16)]
                els = plsc.load_gather(elv, [s16])
                erd = plsc.load_gather(erv, [d16])
                e = els + erd
                e = jnp.where(e >= 0, e, NEG_SLOPE * e)
                ex = jnp.exp(e)
                plsc.addupdate_scatter(denom, [d16], ex)
                for c in range(4):
                    fv = plsc.load_gather(ft, [ccs[c], s16])
                    plsc.addupdate_scatter(acc, [ccs[c], d16], fv * ex)

        start(0, sv0, dv0, sm0, sm1)
        start(1, sv1, dv1, sm2, sm3)

        def pair_body(kk, _):
            ci = kk * 2
            wait(ci, sv0, dv0, sm0, sm1)
            process(sv0, dv0)
            start(ci + 2, sv0, dv0, sm0, sm1)
            wait(ci + 1, sv1, dv1, sm2, sm3)
            process(sv1, dv1)
            start(ci + 3, sv1, dv1, sm2, sm3)
            return 0
        lax.fori_loop(0, NCH // 2, pair_body, 0)
        wait(NCH - 1, sv0, dv0, sm0, sm1)
        wait(NCH - 1, sv1, dv1, sm2, sm3)

        one = jnp.ones((16,), jnp.float32)

        @plsc.parallel_loop(0, NP // 16, unroll=8)
        def _(i):
            dn = denom[pl.ds(i * 16, 16)]
            rd = jnp.where(dn > 0, one / dn, zero)
            for c in range(4):
                acc[c, pl.ds(i * 16, 16)] = acc[c, pl.ds(i * 16, 16)] * rd
        pltpu.sync_copy(acc, out.at[pl.ds(4 * t, 4), :])

    return k


def _final_body(u0_ref, u1_ref, b0, b1, eye, wi, bi, wr1, br1, g, bl, a1,
                wr2, br2, wo, bo_r, a2, out_ref, cat_ref):
    u0 = u0_ref[...] + b0[...]
    u1 = u1_ref[...] + b1[...]
    t0 = _mlp_core_t(u1, wi[...], bi[...], wr1[...], br1[...], g[...],
                     bl[...], a1[...], wr2[...], br2[...])
    o = _dot(t0, wo[...], ((0,), (1,))) + bo_r[...]
    out_ref[...] = _prelu(o, a2[...])
    ident = eye[...]
    u0_rm = _dot(u0, ident, ((0,), (0,)))
    u1_rm = _dot(u1, ident, ((0,), (0,)))
    cat_ref[...] = jnp.concatenate([u0_rm, u1_rm], axis=1)


def _final(a_u0, a_u1, b0col, b1col, p):
    dh = 128
    specs = [
        pl.BlockSpec((D, BN), lambda i: (0, i)),
        pl.BlockSpec((D, BN), lambda i: (0, i)),
        _full((D, 1)), _full((D, 1)), _full((D, D)),
        _full((dh, D)), _full((dh, 1)),
        _full((dh, dh)), _full((dh, 1)),
        _full((dh, 1)), _full((dh, 1)), _full((1, 1)),
        _full((dh, dh)), _full((dh, 1)),
        _full((dh, dh)), _full((1, dh)), _full((1, 1)),
    ]
    fn = pl.pallas_call(
        _final_body,
        grid=(GRID,),
        in_specs=specs,
        out_specs=[pl.BlockSpec((BN, dh), lambda i: (i, 0)),
                   pl.BlockSpec((BN, 2 * D), lambda i: (i, 0))],
        out_shape=[jax.ShapeDtypeStruct((NP, dh), jnp.float32),
                   jax.ShapeDtypeStruct((NP, 2 * D), jnp.float32)],
    )
    return fn(a_u0, a_u1, b0col, b1col, jnp.eye(D, dtype=jnp.float32),
              p['in_w'], p['in_b'].reshape(dh, 1),
              p['r1_w'], p['r1_b'].reshape(dh, 1),
              p['ln_g'].reshape(dh, 1), p['ln_b'].reshape(dh, 1),
              p['a1'].reshape(1, 1),
              p['r2_w'], p['r2_b'].reshape(dh, 1),
              p['out_w'], p['out_b'].reshape(1, dh),
              p['a2'].reshape(1, 1))


_prelude_1_4 = _make_prelude(1, 4)
_prelude_2_4 = _make_prelude(2, 4)
_prelude_2_1 = _make_prelude(2, 1)
_gat_sc_4 = _make_gat_sc(4)
_gat_sc_1 = _make_gat_sc(1)


def kernel(x_t, edge_index, time_embed, params):
    del time_embed
    p = params
    x = jnp.pad(x_t, ((0, NP - N), (0, 0)))
    src = edge_index[0]
    dst = edge_index[1]

    def bcol(q):
        return q['bias'].reshape(D, 1)

    def attn(q):
        return (q['fc_w'], _block_diag_attn(q['attn_l']),
                _block_diag_attn(q['attn_r']))

    zcol = jnp.zeros((D, 1), jnp.float32)

    hT = _mlp_in(x, p['mlp_in'])
    f, el, er = _prelude_1_4(hT, zcol, *attn(p['down0']))
    a0 = _gat_sc_4(f, el, er, src, dst)
    f, el, er = _prelude_1_4(a0, bcol(p['down0']), *attn(p['down1']))
    a1 = _gat_sc_4(f, el, er, src, dst)
    mT = _mlp_mid(a1, bcol(p['down1']), p['mlp_mid'])
    f, el, er = _prelude_2_4(mT, a1, bcol(p['down1']), *attn(p['up0']))
    au0 = _gat_sc_4(f, el, er, src, dst)
    bsum = (p['up0']['bias'] + p['down0']['bias']).reshape(D, 1)
    f, el, er = _prelude_2_1(au0, a0, bsum, *attn(p['up1']))
    au1 = _gat_sc_1(f, el, er, src, dst)
    out_full, cat_full = _final(au0, au1, bcol(p['up0']), bcol(p['up1']),
                                p['mlp_out'])
    return out_full[:N], cat_full[:N]
